# baseline (speedup 1.0000x reference)
import functools

import jax
import jax.numpy as jnp
from jax.experimental import pallas as pl
from jax.experimental.pallas import tpu as pltpu

_BM = 256
_BK = 512


def _gtconv_body(h_ref, s_ref, x_ref, o_ref):
    k = pl.program_id(1)
    hb = h_ref[0, 0] * s_ref[0]
    for i in range(1, s_ref.shape[0]):
        hb = hb + h_ref[0, i] * s_ref[i]
    xk = x_ref[pl.ds(k * _BK, _BK), :]
    part = jnp.dot(hb.astype(jnp.bfloat16), xk,
                   preferred_element_type=jnp.float32)

    @pl.when(k == 0)
    def _init():
        o_ref[...] = part

    @pl.when(k != 0)
    def _acc():
        o_ref[...] += part


@jax.jit
def kernel(x, S_powers, h):
    K, N, _ = S_powers.shape
    D = x.shape[1]
    grid = (N // _BM, N // _BK)
    return pl.pallas_call(
        _gtconv_body,
        grid=grid,
        in_specs=[
            pl.BlockSpec((1, K), lambda i, k: (0, 0)),
            pl.BlockSpec((K, _BM, _BK), lambda i, k: (0, i, k)),
            pl.BlockSpec((N, D), lambda i, k: (0, 0)),
        ],
        out_specs=pl.BlockSpec((_BM, D), lambda i, k: (i, 0)),
        out_shape=jax.ShapeDtypeStruct((N, D), jnp.float32),
        compiler_params=pltpu.CompilerParams(
            dimension_semantics=("parallel", "arbitrary"),
        ),
    )(h.reshape(1, K), S_powers, x.astype(jnp.bfloat16))


# speedup vs baseline: 1.4006x; 1.4006x over previous
import functools

import jax
import jax.numpy as jnp
from jax.experimental import pallas as pl
from jax.experimental.pallas import tpu as pltpu

_BM = 256
_BK = 1024


def _gtconv_body(h_ref, s_ref, x_ref, o_ref):
    k = pl.program_id(1)
    hb = h_ref[0, 0] * s_ref[0]
    for i in range(1, s_ref.shape[0]):
        hb = hb + h_ref[0, i] * s_ref[i]
    xk = x_ref[pl.ds(k * _BK, _BK), :]
    part = jnp.dot(hb.astype(jnp.bfloat16), xk,
                   preferred_element_type=jnp.float32)

    @pl.when(k == 0)
    def _init():
        o_ref[...] = part

    @pl.when(k != 0)
    def _acc():
        o_ref[...] += part


@jax.jit
def kernel(x, S_powers, h):
    K, N, _ = S_powers.shape
    D = x.shape[1]
    grid = (N // _BM, N // _BK)
    return pl.pallas_call(
        _gtconv_body,
        grid=grid,
        in_specs=[
            pl.BlockSpec((1, K), lambda i, k: (0, 0)),
            pl.BlockSpec((K, _BM, _BK), lambda i, k: (0, i, k)),
            pl.BlockSpec((N, D), lambda i, k: (0, 0)),
        ],
        out_specs=pl.BlockSpec((_BM, D), lambda i, k: (i, 0)),
        out_shape=jax.ShapeDtypeStruct((N, D), jnp.float32),
        compiler_params=pltpu.CompilerParams(
            dimension_semantics=("parallel", "arbitrary"),
        ),
    )(h.reshape(1, K), S_powers, x.astype(jnp.bfloat16))


# speedup vs baseline: 1.5234x; 1.0877x over previous
import functools

import jax
import jax.numpy as jnp
from jax.experimental import pallas as pl
from jax.experimental.pallas import tpu as pltpu

_BM = 256
_BK = 2048


def _gtconv_body(h_ref, s_ref, x_ref, o_ref):
    k = pl.program_id(1)
    hb = h_ref[0, 0] * s_ref[0]
    for i in range(1, s_ref.shape[0]):
        hb = hb + h_ref[0, i] * s_ref[i]
    xk = x_ref[pl.ds(k * _BK, _BK), :]
    part = jnp.dot(hb.astype(jnp.bfloat16), xk,
                   preferred_element_type=jnp.float32)

    @pl.when(k == 0)
    def _init():
        o_ref[...] = part

    @pl.when(k != 0)
    def _acc():
        o_ref[...] += part


@jax.jit
def kernel(x, S_powers, h):
    K, N, _ = S_powers.shape
    D = x.shape[1]
    grid = (N // _BM, N // _BK)
    return pl.pallas_call(
        _gtconv_body,
        grid=grid,
        in_specs=[
            pl.BlockSpec((1, K), lambda i, k: (0, 0)),
            pl.BlockSpec((K, _BM, _BK), lambda i, k: (0, i, k)),
            pl.BlockSpec((N, D), lambda i, k: (0, 0)),
        ],
        out_specs=pl.BlockSpec((_BM, D), lambda i, k: (i, 0)),
        out_shape=jax.ShapeDtypeStruct((N, D), jnp.float32),
        compiler_params=pltpu.CompilerParams(
            dimension_semantics=("parallel", "arbitrary"),
        ),
    )(h.reshape(1, K), S_powers, x.astype(jnp.bfloat16))


# speedup vs baseline: 1.5515x; 1.0184x over previous
import functools

import jax
import jax.numpy as jnp
from jax.experimental import pallas as pl
from jax.experimental.pallas import tpu as pltpu

_BM = 256
_BK = 4096


def _gtconv_body(h_ref, s_ref, x_ref, o_ref):
    k = pl.program_id(1)
    hb = h_ref[0, 0] * s_ref[0]
    for i in range(1, s_ref.shape[0]):
        hb = hb + h_ref[0, i] * s_ref[i]
    xk = x_ref[pl.ds(k * _BK, _BK), :]
    part = jnp.dot(hb.astype(jnp.bfloat16), xk,
                   preferred_element_type=jnp.float32)

    @pl.when(k == 0)
    def _init():
        o_ref[...] = part

    @pl.when(k != 0)
    def _acc():
        o_ref[...] += part


@jax.jit
def kernel(x, S_powers, h):
    K, N, _ = S_powers.shape
    D = x.shape[1]
    grid = (N // _BM, N // _BK)
    return pl.pallas_call(
        _gtconv_body,
        grid=grid,
        in_specs=[
            pl.BlockSpec((1, K), lambda i, k: (0, 0)),
            pl.BlockSpec((K, _BM, _BK), lambda i, k: (0, i, k)),
            pl.BlockSpec((N, D), lambda i, k: (0, 0)),
        ],
        out_specs=pl.BlockSpec((_BM, D), lambda i, k: (i, 0)),
        out_shape=jax.ShapeDtypeStruct((N, D), jnp.float32),
        compiler_params=pltpu.CompilerParams(
            dimension_semantics=("parallel", "arbitrary"),
        ),
    )(h.reshape(1, K), S_powers, x.astype(jnp.bfloat16))
